# baseline (speedup 1.0000x reference)
import jax
import jax.numpy as jnp
from jax.experimental import pallas as pl
from jax.experimental.pallas import tpu as pltpu

HIDDEN = 2048
NUM_EXPERTS = 16
TOP_K = 2
TM = 512


def _router_kernel(x_ref, w1_ref, b1_ref, w2_ref, b2_ref,
                   idx_ref, probs_ref, aux_ref, acc_ref):
    i = pl.program_id(0)
    nsteps = pl.num_programs(0)

    h = jnp.dot(x_ref[...], w1_ref[...], preferred_element_type=jnp.float32)
    h = jnp.maximum(h + b1_ref[...], 0.0)
    logits = jnp.dot(h.astype(jnp.bfloat16), w2_ref[...],
                     preferred_element_type=jnp.float32)
    logits = logits + b2_ref[...]

    m1 = jnp.max(logits, axis=-1, keepdims=True)
    e = jnp.exp(logits - m1)
    z = jnp.sum(e, axis=-1, keepdims=True)
    p = e / z

    @pl.when(i == 0)
    def _init():
        acc_ref[...] = jnp.zeros_like(acc_ref)

    acc_ref[...] += jnp.sum(p, axis=0, keepdims=True)

    lane = jax.lax.broadcasted_iota(jnp.int32, logits.shape, 1)
    i1 = jnp.min(jnp.where(logits == m1, lane, NUM_EXPERTS),
                 axis=-1, keepdims=True)
    masked = jnp.where(lane == i1, -jnp.inf, logits)
    m2 = jnp.max(masked, axis=-1, keepdims=True)
    i2 = jnp.min(jnp.where(masked == m2, lane, NUM_EXPERTS),
                 axis=-1, keepdims=True)

    p1 = jnp.sum(jnp.where(lane == i1, p, 0.0), axis=-1, keepdims=True)
    p2 = jnp.sum(jnp.where(lane == i2, p, 0.0), axis=-1, keepdims=True)
    s = p1 + p2
    idx_ref[...] = jnp.concatenate([i1, i2], axis=-1)
    probs_ref[...] = jnp.concatenate([p1 / s, p2 / s], axis=-1)

    @pl.when(i == nsteps - 1)
    def _finalize():
        mean = acc_ref[...] / jnp.float32(nsteps * TM)
        aux_ref[...] = jnp.sum(mean * jnp.log(mean * NUM_EXPERTS + 1e-9)
                               ).reshape(1, 1)


def kernel(x, W1, b1, W2, b2):
    B, S, H = x.shape
    M = B * S
    x2 = x.reshape(M, H).astype(jnp.bfloat16)
    w1 = W1.astype(jnp.bfloat16)
    w2 = W2.astype(jnp.bfloat16)
    b1r = b1.reshape(1, H)
    b2r = b2.reshape(1, NUM_EXPERTS)
    nsteps = M // TM

    idx, probs, aux = pl.pallas_call(
        _router_kernel,
        grid=(nsteps,),
        in_specs=[
            pl.BlockSpec((TM, H), lambda i: (i, 0)),
            pl.BlockSpec((H, H), lambda i: (0, 0)),
            pl.BlockSpec((1, H), lambda i: (0, 0)),
            pl.BlockSpec((H, NUM_EXPERTS), lambda i: (0, 0)),
            pl.BlockSpec((1, NUM_EXPERTS), lambda i: (0, 0)),
        ],
        out_specs=[
            pl.BlockSpec((TM, TOP_K), lambda i: (i, 0)),
            pl.BlockSpec((TM, TOP_K), lambda i: (i, 0)),
            pl.BlockSpec((1, 1), lambda i: (0, 0)),
        ],
        out_shape=[
            jax.ShapeDtypeStruct((M, TOP_K), jnp.int32),
            jax.ShapeDtypeStruct((M, TOP_K), jnp.float32),
            jax.ShapeDtypeStruct((1, 1), jnp.float32),
        ],
        scratch_shapes=[pltpu.VMEM((1, NUM_EXPERTS), jnp.float32)],
        compiler_params=pltpu.CompilerParams(
            dimension_semantics=("arbitrary",),
        ),
    )(x2, w1, b1r, w2, b2r)

    return (idx.reshape(B, S, TOP_K), probs.reshape(B, S, TOP_K),
            aux.reshape(()))


# speedup vs baseline: 1.0105x; 1.0105x over previous
import jax
import jax.numpy as jnp
from jax.experimental import pallas as pl
from jax.experimental.pallas import tpu as pltpu

HIDDEN = 2048
NUM_EXPERTS = 16
TOP_K = 2
TM = 512


def _router_kernel(x_ref, w1_ref, b1_ref, w2_ref, b2_ref,
                   idx_ref, probs_ref, psum_ref):
    h = jnp.dot(x_ref[...], w1_ref[...], preferred_element_type=jnp.float32)
    h = jnp.maximum(h + b1_ref[...], 0.0)
    logits = jnp.dot(h.astype(jnp.bfloat16), w2_ref[...],
                     preferred_element_type=jnp.float32)
    logits = logits + b2_ref[...]

    m1 = jnp.max(logits, axis=-1, keepdims=True)
    e = jnp.exp(logits - m1)
    z = jnp.sum(e, axis=-1, keepdims=True)
    p = e / z

    psum_ref[...] = jnp.sum(p, axis=0, keepdims=True)[None]

    lane = jax.lax.broadcasted_iota(jnp.int32, logits.shape, 1)
    i1 = jnp.min(jnp.where(logits == m1, lane, NUM_EXPERTS),
                 axis=-1, keepdims=True)
    masked = jnp.where(lane == i1, -jnp.inf, logits)
    m2 = jnp.max(masked, axis=-1, keepdims=True)
    i2 = jnp.min(jnp.where(masked == m2, lane, NUM_EXPERTS),
                 axis=-1, keepdims=True)

    p1 = jnp.sum(jnp.where(lane == i1, p, 0.0), axis=-1, keepdims=True)
    p2 = jnp.sum(jnp.where(lane == i2, p, 0.0), axis=-1, keepdims=True)
    s = p1 + p2
    idx_ref[...] = jnp.concatenate([i1, i2], axis=-1)
    probs_ref[...] = jnp.concatenate([p1 / s, p2 / s], axis=-1)


def _aux_kernel(psum_ref, aux_ref, *, total):
    mean = jnp.sum(psum_ref[...], axis=0) / jnp.float32(total)
    aux_ref[...] = jnp.sum(mean * jnp.log(mean * NUM_EXPERTS + 1e-9)
                           ).reshape(1, 1)


def kernel(x, W1, b1, W2, b2):
    B, S, H = x.shape
    M = B * S
    x2 = x.reshape(M, H).astype(jnp.bfloat16)
    w1 = W1.astype(jnp.bfloat16)
    w2 = W2.astype(jnp.bfloat16)
    b1r = b1.reshape(1, H)
    b2r = b2.reshape(1, NUM_EXPERTS)
    nsteps = M // TM

    idx, probs, psums = pl.pallas_call(
        _router_kernel,
        grid=(nsteps,),
        in_specs=[
            pl.BlockSpec((TM, H), lambda i: (i, 0)),
            pl.BlockSpec((H, H), lambda i: (0, 0)),
            pl.BlockSpec((1, H), lambda i: (0, 0)),
            pl.BlockSpec((H, NUM_EXPERTS), lambda i: (0, 0)),
            pl.BlockSpec((1, NUM_EXPERTS), lambda i: (0, 0)),
        ],
        out_specs=[
            pl.BlockSpec((TM, TOP_K), lambda i: (i, 0)),
            pl.BlockSpec((TM, TOP_K), lambda i: (i, 0)),
            pl.BlockSpec((1, 1, NUM_EXPERTS), lambda i: (i, 0, 0)),
        ],
        out_shape=[
            jax.ShapeDtypeStruct((M, TOP_K), jnp.int32),
            jax.ShapeDtypeStruct((M, TOP_K), jnp.float32),
            jax.ShapeDtypeStruct((nsteps, 1, NUM_EXPERTS), jnp.float32),
        ],
        compiler_params=pltpu.CompilerParams(
            dimension_semantics=("parallel",),
        ),
    )(x2, w1, b1r, w2, b2r)

    import functools
    aux = pl.pallas_call(
        functools.partial(_aux_kernel, total=M),
        out_shape=jax.ShapeDtypeStruct((1, 1), jnp.float32),
    )(psums.reshape(nsteps, NUM_EXPERTS))

    return (idx.reshape(B, S, TOP_K), probs.reshape(B, S, TOP_K),
            aux.reshape(()))


# speedup vs baseline: 1.4098x; 1.3952x over previous
import functools

import jax
import jax.numpy as jnp
from jax.experimental import pallas as pl
from jax.experimental.pallas import tpu as pltpu

HIDDEN = 2048
NUM_EXPERTS = 16
TOP_K = 2
TM = 1024


def _router_kernel(x_ref, w1_ref, b1_ref, w2t_ref, b2t_ref,
                   idx_ref, probs_ref, psum_ref):
    h = jnp.dot(x_ref[...].astype(jnp.bfloat16), w1_ref[...],
                preferred_element_type=jnp.float32)
    h = jnp.maximum(h + b1_ref[...], 0.0)
    lt = jax.lax.dot_general(w2t_ref[...], h.astype(jnp.bfloat16),
                             (((1,), (1,)), ((), ())),
                             preferred_element_type=jnp.float32)
    lt = lt + b2t_ref[...]

    m1 = jnp.max(lt, axis=0, keepdims=True)
    e = jnp.exp(lt - m1)
    z = jnp.sum(e, axis=0, keepdims=True)
    p = e / z

    psum_ref[...] = jnp.sum(p, axis=1).reshape(1, 1, NUM_EXPERTS)

    sub = jax.lax.broadcasted_iota(jnp.int32, lt.shape, 0)
    i1 = jnp.min(jnp.where(lt == m1, sub, NUM_EXPERTS),
                 axis=0, keepdims=True)
    masked = jnp.where(sub == i1, -jnp.inf, lt)
    m2 = jnp.max(masked, axis=0, keepdims=True)
    i2 = jnp.min(jnp.where(masked == m2, sub, NUM_EXPERTS),
                 axis=0, keepdims=True)

    p1 = jnp.sum(jnp.where(sub == i1, p, 0.0), axis=0, keepdims=True)
    p2 = jnp.sum(jnp.where(sub == i2, p, 0.0), axis=0, keepdims=True)
    s = p1 + p2
    idx_ref[...] = jnp.concatenate([i1, i2], axis=0).T
    probs_ref[...] = jnp.concatenate([p1 / s, p2 / s], axis=0).T


def _aux_kernel(psum_ref, aux_ref, *, total):
    mean = jnp.sum(psum_ref[...], axis=0) / jnp.float32(total)
    aux_ref[...] = jnp.sum(mean * jnp.log(mean * NUM_EXPERTS + 1e-9)
                           ).reshape(1, 1)


def kernel(x, W1, b1, W2, b2):
    B, S, H = x.shape
    M = B * S
    x2 = x.reshape(M, H)
    w1 = W1.astype(jnp.bfloat16)
    w2t = W2.T.astype(jnp.bfloat16)
    b1r = b1.reshape(1, H)
    b2t = b2.reshape(NUM_EXPERTS, 1)
    nsteps = M // TM

    idx, probs, psums = pl.pallas_call(
        _router_kernel,
        grid=(nsteps,),
        in_specs=[
            pl.BlockSpec((TM, H), lambda i: (i, 0)),
            pl.BlockSpec((H, H), lambda i: (0, 0)),
            pl.BlockSpec((1, H), lambda i: (0, 0)),
            pl.BlockSpec((NUM_EXPERTS, H), lambda i: (0, 0)),
            pl.BlockSpec((NUM_EXPERTS, 1), lambda i: (0, 0)),
        ],
        out_specs=[
            pl.BlockSpec((TM, TOP_K), lambda i: (i, 0)),
            pl.BlockSpec((TM, TOP_K), lambda i: (i, 0)),
            pl.BlockSpec((1, 1, NUM_EXPERTS), lambda i: (i, 0, 0)),
        ],
        out_shape=[
            jax.ShapeDtypeStruct((M, TOP_K), jnp.int32),
            jax.ShapeDtypeStruct((M, TOP_K), jnp.float32),
            jax.ShapeDtypeStruct((nsteps, 1, NUM_EXPERTS), jnp.float32),
        ],
        compiler_params=pltpu.CompilerParams(
            dimension_semantics=("parallel",),
        ),
    )(x2, w1, b1r, w2t, b2t)

    aux = pl.pallas_call(
        functools.partial(_aux_kernel, total=M),
        out_shape=jax.ShapeDtypeStruct((1, 1), jnp.float32),
    )(psums.reshape(nsteps, NUM_EXPERTS))

    return (idx.reshape(B, S, TOP_K), probs.reshape(B, S, TOP_K),
            aux.reshape(()))
